# baseline (speedup 1.0000x reference)
import functools

import jax
import jax.numpy as jnp
from jax import lax
from jax.experimental import pallas as pl
from jax.experimental.pallas import tpu as pltpu
from jax.experimental.pallas import tpu_sc as plsc

NC, NS = 2, 16
NW = NC * NS
BATCH = 16384
D = 64
DF = 2 * D
BPW = BATCH // NW
CHUNK = 32
NCH = BPW // CHUNK


def _sc_gather(users2, items2, utab, itab):
  mesh = plsc.VectorSubcoreMesh(core_axis_name="c", subcore_axis_name="s")
  out_t = [jax.ShapeDtypeStruct((BATCH, DF), jnp.float32) for _ in range(2)]

  @functools.partial(
      pl.kernel,
      mesh=mesh,
      out_type=out_t,
      scratch_types=[
          pltpu.VMEM((CHUNK, DF), jnp.float32),
          pltpu.VMEM((CHUNK, DF), jnp.float32),
          pltpu.VMEM((2 * BPW,), jnp.int32),
          pltpu.SemaphoreType.DMA,
          pltpu.SemaphoreType.DMA,
      ],
  )
  def k(users_hbm, items_hbm, utab_hbm, itab_hbm, o_u, o_i,
        buf0, buf1, idx_vmem, sem0, sem1):
    wid = lax.axis_index("s") * NC + lax.axis_index("c")
    base = wid * BPW
    pltpu.sync_copy(users_hbm.at[wid], idx_vmem.at[pl.ds(0, BPW)])
    pltpu.sync_copy(items_hbm.at[wid], idx_vmem.at[pl.ds(BPW, BPW)])

    bufs = (buf0, buf1)
    sems = (sem0, sem1)

    def do_table(table_hbm, off, out_hbm):
      def fire(chunk, sl):
        for p in range(CHUNK // 16):
          vec = idx_vmem[pl.ds(off + chunk * CHUNK + 16 * p, 16)]
          for e in range(16):
            pltpu.make_async_copy(
                table_hbm.at[vec[e]], bufs[sl].at[16 * p + e],
                sems[sl]).start()

      def drain(chunk, sl):
        for e in range(CHUNK):
          pltpu.make_async_copy(
              table_hbm.at[0], bufs[sl].at[e], sems[sl]).wait()

      def flush(chunk, sl):
        pltpu.sync_copy(bufs[sl],
                        out_hbm.at[pl.ds(base + chunk * CHUNK, CHUNK)])

      fire(0, 0)

      def body(i, carry):
        fire(2 * i + 1, 1)
        drain(2 * i, 0)
        flush(2 * i, 0)

        @pl.when(i < NCH // 2 - 1)
        def _():
          fire(2 * i + 2, 0)

        drain(2 * i + 1, 1)
        flush(2 * i + 1, 1)
        return carry

      lax.fori_loop(0, NCH // 2, body, 0)

    do_table(utab_hbm, 0, o_u)
    do_table(itab_hbm, BPW, o_i)

  return k(users2, items2, utab, itab)


FV = 512
NV = 1000000


def _fuse_body(aT, bT, out):
  a = jnp.transpose(aT[:], (1, 0))
  b = jnp.transpose(bT[:], (1, 0))
  out[:] = jnp.concatenate([a, b], axis=1)


def _tc_fuse(aT, bT):
  colsv = pl.BlockSpec((D, FV), lambda i: (0, i))
  return pl.pallas_call(
      _fuse_body,
      grid=(pl.cdiv(NV, FV),),
      in_specs=[colsv, colsv],
      out_specs=pl.BlockSpec((FV, DF), lambda i: (i, 0)),
      out_shape=jax.ShapeDtypeStruct((NV, DF), jnp.float32),
  )(aT, bT)


BLK = 1024


def _tc_body(ur, ir, w1u, w1i, b1r, w2, b2r, wfm, wfh, bfr, out):
  h = lax.dot_general(ur[:], w1u[:], (((1,), (1,)), ((), ())),
                      preferred_element_type=jnp.float32)
  h = h + lax.dot_general(ir[:], w1i[:], (((1,), (1,)), ((), ())),
                          preferred_element_type=jnp.float32)
  h = jnp.maximum(h + b1r[:], 0.0)
  h = lax.dot_general(h, w2[:], (((1,), (1,)), ((), ())),
                      preferred_element_type=jnp.float32)
  h = jnp.maximum(h + b2r[:], 0.0)
  pm = ur[:] * ir[:]
  out[:] = (jnp.sum(pm * wfm[:], axis=1, keepdims=True)
            + jnp.sum(h * wfh[:], axis=1, keepdims=True) + bfr[0, 0])


def _tc_mlp(u_rows, i_rows, w1u, w1i, b1, w2, b2, wfm, wfh, bf):
  rows = pl.BlockSpec((BLK, DF), lambda i: (i, 0))
  full = lambda a: pl.BlockSpec(a.shape, lambda i: tuple(0 for _ in a.shape))
  return pl.pallas_call(
      _tc_body,
      grid=(BATCH // BLK,),
      in_specs=[rows, rows, full(w1u), full(w1i), full(b1), full(w2),
                full(b2), full(wfm), full(wfh), full(bf)],
      out_specs=pl.BlockSpec((BLK, 1), lambda i: (i, 0)),
      out_shape=jax.ShapeDtypeStruct((BATCH, 1), jnp.float32),
  )(u_rows, i_rows, w1u, w1i, b1, w2, b2, wfm, wfh, bf)


def kernel(users, items, user_mf, item_mf, user_mlp, item_mlp,
           W1, b1, W2, b2, Wf, bf):
  u2 = users.reshape(NW, BPW)
  i2 = items.reshape(NW, BPW)
  utab = _tc_fuse(user_mf.T, user_mlp.T)
  itab = _tc_fuse(item_mf.T, item_mlp.T)
  u_rows, i_rows = _sc_gather(u2, i2, utab, itab)
  zeros = jnp.zeros((W1.shape[0], D), jnp.float32)
  w1u = jnp.concatenate([zeros, W1[:, :D]], axis=1)
  w1i = jnp.concatenate([zeros, W1[:, D:]], axis=1)
  wfm = jnp.concatenate([Wf[:, :D], jnp.zeros((1, D), jnp.float32)],
                        axis=1)
  wfh = Wf[:, D:]
  return _tc_mlp(u_rows, i_rows, w1u, w1i, b1.reshape(1, -1),
                 W2, b2.reshape(1, -1), wfm, wfh, bf.reshape(1, 1))


# speedup vs baseline: 2.5081x; 2.5081x over previous
import functools

import jax
import jax.numpy as jnp
from jax import lax
from jax.experimental import pallas as pl
from jax.experimental.pallas import tpu as pltpu
from jax.experimental.pallas import tpu_sc as plsc

NC, NS = 2, 16
NW = NC * NS
BATCH = 16384
D = 64
DF = 2 * D
BPW = BATCH // NW
CHUNK = 128
NCH = BPW // CHUNK


def _sc_gather(users3, items3, utab, itab):
  mesh = plsc.VectorSubcoreMesh(core_axis_name="c", subcore_axis_name="s")
  out_t = [jax.ShapeDtypeStruct((BATCH, DF), jnp.float32) for _ in range(2)]

  @functools.partial(
      pl.kernel,
      mesh=mesh,
      out_type=out_t,
      scratch_types=[
          pltpu.VMEM((NCH, CHUNK), jnp.int32),
          pltpu.VMEM((NCH, CHUNK), jnp.int32),
          pltpu.VMEM((CHUNK, DF), jnp.float32),
          pltpu.VMEM((CHUNK, DF), jnp.float32),
          pltpu.SemaphoreType.DMA,
          pltpu.SemaphoreType.DMA,
      ],
  )
  def k(users_hbm, items_hbm, utab_hbm, itab_hbm, o_u, o_i,
        idx_u, idx_i, buf0, buf1, sem0, sem1):
    wid = lax.axis_index("s") * NC + lax.axis_index("c")
    base = wid * BPW
    pltpu.sync_copy(users_hbm.at[wid], idx_u)
    pltpu.sync_copy(items_hbm.at[wid], idx_i)

    bufs = (buf0, buf1)
    sems = (sem0, sem1)
    plan = [(utab_hbm, idx_u, o_u, c) for c in range(NCH)] + [
        (itab_hbm, idx_i, o_i, c) for c in range(NCH)]

    def fire(j):
      tab, idx, _, c = plan[j]
      pltpu.async_copy(tab.at[idx.at[c]], bufs[j % 2], sems[j % 2])

    def finish(j):
      tab, idx, out, c = plan[j]
      pltpu.make_async_copy(tab.at[idx.at[c]], bufs[j % 2],
                            sems[j % 2]).wait()
      pltpu.sync_copy(bufs[j % 2],
                      out.at[pl.ds(base + c * CHUNK, CHUNK)])

    fire(0)
    for j in range(1, 2 * NCH):
      fire(j)
      finish(j - 1)
    finish(2 * NCH - 1)

  return k(users3, items3, utab, itab)


FV = 2048
NV = 1000000


def _fuse_body(aT, bT, eye, out):
  ab = jnp.concatenate([aT[:], bT[:]], axis=0)
  out[:] = lax.dot_general(ab, eye[:], (((0,), (0,)), ((), ())),
                           preferred_element_type=jnp.float32)


def _tc_fuse(aT, bT, eye):
  colsv = pl.BlockSpec((D, FV), lambda i: (0, i))
  return pl.pallas_call(
      _fuse_body,
      grid=(pl.cdiv(NV, FV),),
      in_specs=[colsv, colsv, pl.BlockSpec((DF, DF), lambda i: (0, 0))],
      out_specs=pl.BlockSpec((FV, DF), lambda i: (i, 0)),
      out_shape=jax.ShapeDtypeStruct((NV, DF), jnp.float32),
  )(aT, bT, eye)


BLK = 1024


def _tc_body(ur, ir, w1u, w1i, b1r, w2, b2r, wfm, wfh, bfr, out):
  h = lax.dot_general(ur[:], w1u[:], (((1,), (1,)), ((), ())),
                      preferred_element_type=jnp.float32)
  h = h + lax.dot_general(ir[:], w1i[:], (((1,), (1,)), ((), ())),
                          preferred_element_type=jnp.float32)
  h = jnp.maximum(h + b1r[:], 0.0)
  h = lax.dot_general(h, w2[:], (((1,), (1,)), ((), ())),
                      preferred_element_type=jnp.float32)
  h = jnp.maximum(h + b2r[:], 0.0)
  pm = ur[:] * ir[:]
  out[:] = (jnp.sum(pm * wfm[:], axis=1, keepdims=True)
            + jnp.sum(h * wfh[:], axis=1, keepdims=True) + bfr[0, 0])


def _tc_mlp(u_rows, i_rows, w1u, w1i, b1, w2, b2, wfm, wfh, bf):
  rows = pl.BlockSpec((BLK, DF), lambda i: (i, 0))
  full = lambda a: pl.BlockSpec(a.shape, lambda i: tuple(0 for _ in a.shape))
  return pl.pallas_call(
      _tc_body,
      grid=(BATCH // BLK,),
      in_specs=[rows, rows, full(w1u), full(w1i), full(b1), full(w2),
                full(b2), full(wfm), full(wfh), full(bf)],
      out_specs=pl.BlockSpec((BLK, 1), lambda i: (i, 0)),
      out_shape=jax.ShapeDtypeStruct((BATCH, 1), jnp.float32),
  )(u_rows, i_rows, w1u, w1i, b1, w2, b2, wfm, wfh, bf)


def kernel(users, items, user_mf, item_mf, user_mlp, item_mlp,
           W1, b1, W2, b2, Wf, bf):
  u3 = users.reshape(NW, NCH, CHUNK)
  i3 = items.reshape(NW, NCH, CHUNK)
  eye = jnp.eye(DF, dtype=jnp.float32)
  utab = _tc_fuse(user_mf.T, user_mlp.T, eye)
  itab = _tc_fuse(item_mf.T, item_mlp.T, eye)
  u_rows, i_rows = _sc_gather(u3, i3, utab, itab)
  zeros = jnp.zeros((W1.shape[0], D), jnp.float32)
  w1u = jnp.concatenate([zeros, W1[:, :D]], axis=1)
  w1i = jnp.concatenate([zeros, W1[:, D:]], axis=1)
  wfm = jnp.concatenate([Wf[:, :D], jnp.zeros((1, D), jnp.float32)],
                        axis=1)
  wfh = Wf[:, D:]
  return _tc_mlp(u_rows, i_rows, w1u, w1i, b1.reshape(1, -1),
                 W2, b2.reshape(1, -1), wfm, wfh, bf.reshape(1, 1))


# speedup vs baseline: 3.2552x; 1.2979x over previous
import functools

import jax
import jax.numpy as jnp
from jax import lax
from jax.experimental import pallas as pl
from jax.experimental.pallas import tpu as pltpu
from jax.experimental.pallas import tpu_sc as plsc

NC, NS = 2, 16
NW = NC * NS
BATCH = 16384
D = 64
DF = 2 * D
BPW = BATCH // NW
CHUNK = 128
NCH = BPW // CHUNK


def _sc_gather(users3, items3, utab, itab):
  mesh = plsc.VectorSubcoreMesh(core_axis_name="c", subcore_axis_name="s")
  out_t = [jax.ShapeDtypeStruct((BATCH, DF), jnp.float32) for _ in range(2)]

  @functools.partial(
      pl.kernel,
      mesh=mesh,
      out_type=out_t,
      scratch_types=[
          pltpu.VMEM((NCH, CHUNK), jnp.int32),
          pltpu.VMEM((NCH, CHUNK), jnp.int32),
          pltpu.VMEM((CHUNK, DF), jnp.float32),
          pltpu.VMEM((CHUNK, DF), jnp.float32),
          pltpu.SemaphoreType.DMA,
          pltpu.SemaphoreType.DMA,
      ],
  )
  def k(users_hbm, items_hbm, utab_hbm, itab_hbm, o_u, o_i,
        idx_u, idx_i, buf0, buf1, sem0, sem1):
    wid = lax.axis_index("s") * NC + lax.axis_index("c")
    base = wid * BPW
    pltpu.sync_copy(users_hbm.at[wid], idx_u)
    pltpu.sync_copy(items_hbm.at[wid], idx_i)

    bufs = (buf0, buf1)
    sems = (sem0, sem1)
    plan = [(utab_hbm, idx_u, o_u, c) for c in range(NCH)] + [
        (itab_hbm, idx_i, o_i, c) for c in range(NCH)]

    def fire(j):
      tab, idx, _, c = plan[j]
      pltpu.async_copy(tab.at[idx.at[c]], bufs[j % 2], sems[j % 2])

    def finish(j):
      tab, idx, out, c = plan[j]
      pltpu.make_async_copy(tab.at[idx.at[c]], bufs[j % 2],
                            sems[j % 2]).wait()
      pltpu.sync_copy(bufs[j % 2],
                      out.at[pl.ds(base + c * CHUNK, CHUNK)])

    fire(0)
    for j in range(1, 2 * NCH):
      fire(j)
      finish(j - 1)
    finish(2 * NCH - 1)

  return k(users3, items3, utab, itab)


FV = 4096
NV = 1000000


def _fuse_body(aT, bT, eye2, out):
  ab = jnp.concatenate([aT[:], bT[:]], axis=0)
  hi = ab.astype(jnp.bfloat16)
  lo = (ab - hi.astype(jnp.float32)).astype(jnp.bfloat16)
  hilo = jnp.concatenate([hi, lo], axis=0)
  out[:] = lax.dot_general(hilo, eye2[:], (((0,), (0,)), ((), ())),
                           preferred_element_type=jnp.float32)


def _tc_fuse(aT, bT, eye):
  colsv = pl.BlockSpec((D, FV), lambda i: (0, i))
  return pl.pallas_call(
      _fuse_body,
      grid=(pl.cdiv(NV, FV),),
      in_specs=[colsv, colsv,
                pl.BlockSpec((2 * DF, DF), lambda i: (0, 0))],
      out_specs=pl.BlockSpec((FV, DF), lambda i: (i, 0)),
      out_shape=jax.ShapeDtypeStruct((NV, DF), jnp.float32),
  )(aT, bT, eye)


BLK = 1024


def _tc_body(ur, ir, w1u, w1i, b1r, w2, b2r, wfm, wfh, bfr, out):
  h = lax.dot_general(ur[:], w1u[:], (((1,), (1,)), ((), ())),
                      preferred_element_type=jnp.float32)
  h = h + lax.dot_general(ir[:], w1i[:], (((1,), (1,)), ((), ())),
                          preferred_element_type=jnp.float32)
  h = jnp.maximum(h + b1r[:], 0.0)
  h = lax.dot_general(h, w2[:], (((1,), (1,)), ((), ())),
                      preferred_element_type=jnp.float32)
  h = jnp.maximum(h + b2r[:], 0.0)
  pm = ur[:] * ir[:]
  out[:] = (jnp.sum(pm * wfm[:], axis=1, keepdims=True)
            + jnp.sum(h * wfh[:], axis=1, keepdims=True) + bfr[0, 0])


def _tc_mlp(u_rows, i_rows, w1u, w1i, b1, w2, b2, wfm, wfh, bf):
  rows = pl.BlockSpec((BLK, DF), lambda i: (i, 0))
  full = lambda a: pl.BlockSpec(a.shape, lambda i: tuple(0 for _ in a.shape))
  return pl.pallas_call(
      _tc_body,
      grid=(BATCH // BLK,),
      in_specs=[rows, rows, full(w1u), full(w1i), full(b1), full(w2),
                full(b2), full(wfm), full(wfh), full(bf)],
      out_specs=pl.BlockSpec((BLK, 1), lambda i: (i, 0)),
      out_shape=jax.ShapeDtypeStruct((BATCH, 1), jnp.float32),
  )(u_rows, i_rows, w1u, w1i, b1, w2, b2, wfm, wfh, bf)


def kernel(users, items, user_mf, item_mf, user_mlp, item_mlp,
           W1, b1, W2, b2, Wf, bf):
  u3 = users.reshape(NW, NCH, CHUNK)
  i3 = items.reshape(NW, NCH, CHUNK)
  eyeb = jnp.eye(DF, dtype=jnp.bfloat16)
  eye = jnp.concatenate([eyeb, eyeb], axis=0)
  utab = _tc_fuse(user_mf.T, user_mlp.T, eye)
  itab = _tc_fuse(item_mf.T, item_mlp.T, eye)
  u_rows, i_rows = _sc_gather(u3, i3, utab, itab)
  zeros = jnp.zeros((W1.shape[0], D), jnp.float32)
  w1u = jnp.concatenate([zeros, W1[:, :D]], axis=1)
  w1i = jnp.concatenate([zeros, W1[:, D:]], axis=1)
  wfm = jnp.concatenate([Wf[:, :D], jnp.zeros((1, D), jnp.float32)],
                        axis=1)
  wfh = Wf[:, D:]
  return _tc_mlp(u_rows, i_rows, w1u, w1i, b1.reshape(1, -1),
                 W2, b2.reshape(1, -1), wfm, wfh, bf.reshape(1, 1))


# speedup vs baseline: 3.9255x; 1.2059x over previous
import functools

import jax
import jax.numpy as jnp
from jax import lax
from jax.experimental import pallas as pl
from jax.experimental.pallas import tpu as pltpu
from jax.experimental.pallas import tpu_sc as plsc

NC, NS = 2, 16
NW = NC * NS
BATCH = 16384
D = 64
DF = 2 * D
BPW = BATCH // NW
CHUNK = 128
NCH = BPW // CHUNK


def _sc_gather(users3, items3, utab, itab):
  mesh = plsc.VectorSubcoreMesh(core_axis_name="c", subcore_axis_name="s")
  out_t = [jax.ShapeDtypeStruct((BATCH, DF), jnp.float32) for _ in range(2)]

  @functools.partial(
      pl.kernel,
      mesh=mesh,
      out_type=out_t,
      scratch_types=[
          pltpu.VMEM((NCH, CHUNK), jnp.int32),
          pltpu.VMEM((NCH, CHUNK), jnp.int32),
          pltpu.VMEM((CHUNK, DF), jnp.float32),
          pltpu.VMEM((CHUNK, DF), jnp.float32),
          pltpu.SemaphoreType.DMA,
          pltpu.SemaphoreType.DMA,
      ],
  )
  def k(users_hbm, items_hbm, utab_hbm, itab_hbm, o_u, o_i,
        idx_u, idx_i, buf0, buf1, sem0, sem1):
    wid = lax.axis_index("s") * NC + lax.axis_index("c")
    base = wid * BPW
    pltpu.sync_copy(users_hbm.at[wid], idx_u)
    pltpu.sync_copy(items_hbm.at[wid], idx_i)

    bufs = (buf0, buf1)
    sems = (sem0, sem1)
    plan = [(utab_hbm, idx_u, o_u, c) for c in range(NCH)] + [
        (itab_hbm, idx_i, o_i, c) for c in range(NCH)]

    def fire(j):
      tab, idx, _, c = plan[j]
      pltpu.async_copy(tab.at[idx.at[c]], bufs[j % 2], sems[j % 2])

    def finish(j):
      tab, idx, out, c = plan[j]
      pltpu.make_async_copy(tab.at[idx.at[c]], bufs[j % 2],
                            sems[j % 2]).wait()
      pltpu.sync_copy(bufs[j % 2],
                      out.at[pl.ds(base + c * CHUNK, CHUNK)])

    fire(0)
    for j in range(1, 2 * NCH):
      fire(j)
      finish(j - 1)
    finish(2 * NCH - 1)

  return k(users3, items3, utab, itab)


FV = 8192
NV = 1000000


def _fuse_body(aT, bT, eye2, out):
  ab = jnp.concatenate([aT[:], bT[:]], axis=0)
  hi = ab.astype(jnp.bfloat16)
  lo = (ab - hi.astype(jnp.float32)).astype(jnp.bfloat16)
  hilo = jnp.concatenate([hi, lo], axis=0)
  out[:] = lax.dot_general(hilo, eye2[:], (((0,), (0,)), ((), ())),
                           preferred_element_type=jnp.float32)


def _tc_fuse(aT, bT, eye):
  colsv = pl.BlockSpec((D, FV), lambda i: (0, i))
  return pl.pallas_call(
      _fuse_body,
      grid=(pl.cdiv(NV, FV),),
      in_specs=[colsv, colsv,
                pl.BlockSpec((2 * DF, DF), lambda i: (0, 0))],
      out_specs=pl.BlockSpec((FV, DF), lambda i: (i, 0)),
      out_shape=jax.ShapeDtypeStruct((NV, DF), jnp.float32),
  )(aT, bT, eye)


BLK = 1024


def _tc_body(ur, ir, w1u, w1i, b1r, w2, b2r, wfm, wfh, bfr, out):
  h = lax.dot_general(ur[:], w1u[:], (((1,), (1,)), ((), ())),
                      preferred_element_type=jnp.float32)
  h = h + lax.dot_general(ir[:], w1i[:], (((1,), (1,)), ((), ())),
                          preferred_element_type=jnp.float32)
  h = jnp.maximum(h + b1r[:], 0.0)
  h = lax.dot_general(h, w2[:], (((1,), (1,)), ((), ())),
                      preferred_element_type=jnp.float32)
  h = jnp.maximum(h + b2r[:], 0.0)
  pm = ur[:] * ir[:]
  out[:] = (jnp.sum(pm * wfm[:], axis=1, keepdims=True)
            + jnp.sum(h * wfh[:], axis=1, keepdims=True) + bfr[0, 0])


def _tc_mlp(u_rows, i_rows, w1u, w1i, b1, w2, b2, wfm, wfh, bf):
  rows = pl.BlockSpec((BLK, DF), lambda i: (i, 0))
  full = lambda a: pl.BlockSpec(a.shape, lambda i: tuple(0 for _ in a.shape))
  return pl.pallas_call(
      _tc_body,
      grid=(BATCH // BLK,),
      in_specs=[rows, rows, full(w1u), full(w1i), full(b1), full(w2),
                full(b2), full(wfm), full(wfh), full(bf)],
      out_specs=pl.BlockSpec((BLK, 1), lambda i: (i, 0)),
      out_shape=jax.ShapeDtypeStruct((BATCH, 1), jnp.float32),
  )(u_rows, i_rows, w1u, w1i, b1, w2, b2, wfm, wfh, bf)


def kernel(users, items, user_mf, item_mf, user_mlp, item_mlp,
           W1, b1, W2, b2, Wf, bf):
  u3 = users.reshape(NW, NCH, CHUNK)
  i3 = items.reshape(NW, NCH, CHUNK)
  eyeb = jnp.eye(DF, dtype=jnp.bfloat16)
  eye = jnp.concatenate([eyeb, eyeb], axis=0)
  utab = _tc_fuse(user_mf.T, user_mlp.T, eye)
  itab = _tc_fuse(item_mf.T, item_mlp.T, eye)
  u_rows, i_rows = _sc_gather(u3, i3, utab, itab)
  zeros = jnp.zeros((W1.shape[0], D), jnp.float32)
  w1u = jnp.concatenate([zeros, W1[:, :D]], axis=1)
  w1i = jnp.concatenate([zeros, W1[:, D:]], axis=1)
  wfm = jnp.concatenate([Wf[:, :D], jnp.zeros((1, D), jnp.float32)],
                        axis=1)
  wfh = Wf[:, D:]
  return _tc_mlp(u_rows, i_rows, w1u, w1i, b1.reshape(1, -1),
                 W2, b2.reshape(1, -1), wfm, wfh, bf.reshape(1, 1))


# speedup vs baseline: 4.0881x; 1.0414x over previous
import functools

import jax
import jax.numpy as jnp
from jax import lax
from jax.experimental import pallas as pl
from jax.experimental.pallas import tpu as pltpu
from jax.experimental.pallas import tpu_sc as plsc

NC, NS = 2, 16
NW = NC * NS
BATCH = 16384
D = 64
DF = 2 * D
BPW = BATCH // NW
CHUNK = 128
NCH = BPW // CHUNK


def _sc_gather(users3, items3, utab, itab):
  mesh = plsc.VectorSubcoreMesh(core_axis_name="c", subcore_axis_name="s")
  out_t = [jax.ShapeDtypeStruct((BATCH, DF), jnp.float32) for _ in range(2)]

  @functools.partial(
      pl.kernel,
      mesh=mesh,
      out_type=out_t,
      scratch_types=[
          pltpu.VMEM((NCH, CHUNK), jnp.int32),
          pltpu.VMEM((NCH, CHUNK), jnp.int32),
          pltpu.VMEM((CHUNK, DF), jnp.float32),
          pltpu.VMEM((CHUNK, DF), jnp.float32),
          pltpu.SemaphoreType.DMA,
          pltpu.SemaphoreType.DMA,
      ],
  )
  def k(users_hbm, items_hbm, utab_hbm, itab_hbm, o_u, o_i,
        idx_u, idx_i, buf0, buf1, sem0, sem1):
    wid = lax.axis_index("s") * NC + lax.axis_index("c")
    base = wid * BPW
    pltpu.sync_copy(users_hbm.at[wid], idx_u)
    pltpu.sync_copy(items_hbm.at[wid], idx_i)

    bufs = (buf0, buf1)
    sems = (sem0, sem1)
    plan = [(utab_hbm, idx_u, o_u, c) for c in range(NCH)] + [
        (itab_hbm, idx_i, o_i, c) for c in range(NCH)]

    def fire(j):
      tab, idx, _, c = plan[j]
      pltpu.async_copy(tab.at[idx.at[c]], bufs[j % 2], sems[j % 2])

    def finish(j):
      tab, idx, out, c = plan[j]
      pltpu.make_async_copy(tab.at[idx.at[c]], bufs[j % 2],
                            sems[j % 2]).wait()
      pltpu.sync_copy(bufs[j % 2],
                      out.at[pl.ds(base + c * CHUNK, CHUNK)])

    fire(0)
    for j in range(1, 2 * NCH):
      fire(j)
      finish(j - 1)
    finish(2 * NCH - 1)

  return k(users3, items3, utab, itab)


FV = 16384
NV = 1000000


def _fuse_body(aT, bT, eye2, out):
  ab = jnp.concatenate([aT[:], bT[:]], axis=0)
  hi = ab.astype(jnp.bfloat16)
  lo = (ab - hi.astype(jnp.float32)).astype(jnp.bfloat16)
  hilo = jnp.concatenate([hi, lo], axis=0)
  out[:] = lax.dot_general(hilo, eye2[:], (((0,), (0,)), ((), ())),
                           preferred_element_type=jnp.float32)


def _tc_fuse(aT, bT, eye):
  colsv = pl.BlockSpec((D, FV), lambda i: (0, i))
  return pl.pallas_call(
      _fuse_body,
      grid=(pl.cdiv(NV, FV),),
      in_specs=[colsv, colsv,
                pl.BlockSpec((2 * DF, DF), lambda i: (0, 0))],
      out_specs=pl.BlockSpec((FV, DF), lambda i: (i, 0)),
      out_shape=jax.ShapeDtypeStruct((NV, DF), jnp.float32),
  )(aT, bT, eye)


BLK = 1024


def _tc_body(ur, ir, w1u, w1i, b1r, w2, b2r, wfm, wfh, bfr, out):
  h = lax.dot_general(ur[:], w1u[:], (((1,), (1,)), ((), ())),
                      preferred_element_type=jnp.float32)
  h = h + lax.dot_general(ir[:], w1i[:], (((1,), (1,)), ((), ())),
                          preferred_element_type=jnp.float32)
  h = jnp.maximum(h + b1r[:], 0.0)
  h = lax.dot_general(h, w2[:], (((1,), (1,)), ((), ())),
                      preferred_element_type=jnp.float32)
  h = jnp.maximum(h + b2r[:], 0.0)
  pm = ur[:] * ir[:]
  out[:] = (jnp.sum(pm * wfm[:], axis=1, keepdims=True)
            + jnp.sum(h * wfh[:], axis=1, keepdims=True) + bfr[0, 0])


def _tc_mlp(u_rows, i_rows, w1u, w1i, b1, w2, b2, wfm, wfh, bf):
  rows = pl.BlockSpec((BLK, DF), lambda i: (i, 0))
  full = lambda a: pl.BlockSpec(a.shape, lambda i: tuple(0 for _ in a.shape))
  return pl.pallas_call(
      _tc_body,
      grid=(BATCH // BLK,),
      in_specs=[rows, rows, full(w1u), full(w1i), full(b1), full(w2),
                full(b2), full(wfm), full(wfh), full(bf)],
      out_specs=pl.BlockSpec((BLK, 1), lambda i: (i, 0)),
      out_shape=jax.ShapeDtypeStruct((BATCH, 1), jnp.float32),
  )(u_rows, i_rows, w1u, w1i, b1, w2, b2, wfm, wfh, bf)


def kernel(users, items, user_mf, item_mf, user_mlp, item_mlp,
           W1, b1, W2, b2, Wf, bf):
  u3 = users.reshape(NW, NCH, CHUNK)
  i3 = items.reshape(NW, NCH, CHUNK)
  eyeb = jnp.eye(DF, dtype=jnp.bfloat16)
  eye = jnp.concatenate([eyeb, eyeb], axis=0)
  utab = _tc_fuse(user_mf.T, user_mlp.T, eye)
  itab = _tc_fuse(item_mf.T, item_mlp.T, eye)
  u_rows, i_rows = _sc_gather(u3, i3, utab, itab)
  zeros = jnp.zeros((W1.shape[0], D), jnp.float32)
  w1u = jnp.concatenate([zeros, W1[:, :D]], axis=1)
  w1i = jnp.concatenate([zeros, W1[:, D:]], axis=1)
  wfm = jnp.concatenate([Wf[:, :D], jnp.zeros((1, D), jnp.float32)],
                        axis=1)
  wfh = Wf[:, D:]
  return _tc_mlp(u_rows, i_rows, w1u, w1i, b1.reshape(1, -1),
                 W2, b2.reshape(1, -1), wfm, wfh, bf.reshape(1, 1))


# speedup vs baseline: 4.1038x; 1.0038x over previous
import functools

import jax
import jax.numpy as jnp
from jax import lax
from jax.experimental import pallas as pl
from jax.experimental.pallas import tpu as pltpu
from jax.experimental.pallas import tpu_sc as plsc

NC, NS = 2, 16
NW = NC * NS
BATCH = 16384
D = 64
DF = 2 * D
BPW = BATCH // NW
CHUNK = 128
NCH = BPW // CHUNK


def _sc_gather(users3, items3, utab, itab):
  mesh = plsc.VectorSubcoreMesh(core_axis_name="c", subcore_axis_name="s")
  out_t = [jax.ShapeDtypeStruct((BATCH, DF), jnp.float32) for _ in range(2)]

  @functools.partial(
      pl.kernel,
      mesh=mesh,
      out_type=out_t,
      scratch_types=[
          pltpu.VMEM((NCH, CHUNK), jnp.int32),
          pltpu.VMEM((NCH, CHUNK), jnp.int32),
          pltpu.VMEM((CHUNK, DF), jnp.float32),
          pltpu.VMEM((CHUNK, DF), jnp.float32),
          pltpu.SemaphoreType.DMA,
          pltpu.SemaphoreType.DMA,
      ],
  )
  def k(users_hbm, items_hbm, utab_hbm, itab_hbm, o_u, o_i,
        idx_u, idx_i, buf0, buf1, sem0, sem1):
    wid = lax.axis_index("s") * NC + lax.axis_index("c")
    base = wid * BPW
    pltpu.sync_copy(users_hbm.at[wid], idx_u)
    pltpu.sync_copy(items_hbm.at[wid], idx_i)

    bufs = (buf0, buf1)
    sems = (sem0, sem1)
    plan = [(utab_hbm, idx_u, o_u, c) for c in range(NCH)] + [
        (itab_hbm, idx_i, o_i, c) for c in range(NCH)]

    def fire(j):
      tab, idx, _, c = plan[j]
      pltpu.async_copy(tab.at[idx.at[c]], bufs[j % 2], sems[j % 2])

    def finish(j):
      tab, idx, out, c = plan[j]
      pltpu.make_async_copy(tab.at[idx.at[c]], bufs[j % 2],
                            sems[j % 2]).wait()
      pltpu.sync_copy(bufs[j % 2],
                      out.at[pl.ds(base + c * CHUNK, CHUNK)])

    fire(0)
    for j in range(1, 2 * NCH):
      fire(j)
      finish(j - 1)
    finish(2 * NCH - 1)

  return k(users3, items3, utab, itab)


FV = 24576
NV = 1000000


def _fuse_body(aT, bT, eye2, out):
  ab = jnp.concatenate([aT[:], bT[:]], axis=0)
  hi = ab.astype(jnp.bfloat16)
  lo = (ab - hi.astype(jnp.float32)).astype(jnp.bfloat16)
  hilo = jnp.concatenate([hi, lo], axis=0)
  out[:] = lax.dot_general(hilo, eye2[:], (((0,), (0,)), ((), ())),
                           preferred_element_type=jnp.float32)


def _tc_fuse(aT, bT, eye):
  colsv = pl.BlockSpec((D, FV), lambda i: (0, i))
  return pl.pallas_call(
      _fuse_body,
      grid=(pl.cdiv(NV, FV),),
      in_specs=[colsv, colsv,
                pl.BlockSpec((2 * DF, DF), lambda i: (0, 0))],
      out_specs=pl.BlockSpec((FV, DF), lambda i: (i, 0)),
      out_shape=jax.ShapeDtypeStruct((NV, DF), jnp.float32),
  )(aT, bT, eye)


BLK = 1024


def _tc_body(ur, ir, w1u, w1i, b1r, w2, b2r, wfm, wfh, bfr, out):
  h = lax.dot_general(ur[:], w1u[:], (((1,), (1,)), ((), ())),
                      preferred_element_type=jnp.float32)
  h = h + lax.dot_general(ir[:], w1i[:], (((1,), (1,)), ((), ())),
                          preferred_element_type=jnp.float32)
  h = jnp.maximum(h + b1r[:], 0.0)
  h = lax.dot_general(h, w2[:], (((1,), (1,)), ((), ())),
                      preferred_element_type=jnp.float32)
  h = jnp.maximum(h + b2r[:], 0.0)
  pm = ur[:] * ir[:]
  out[:] = (jnp.sum(pm * wfm[:], axis=1, keepdims=True)
            + jnp.sum(h * wfh[:], axis=1, keepdims=True) + bfr[0, 0])


def _tc_mlp(u_rows, i_rows, w1u, w1i, b1, w2, b2, wfm, wfh, bf):
  rows = pl.BlockSpec((BLK, DF), lambda i: (i, 0))
  full = lambda a: pl.BlockSpec(a.shape, lambda i: tuple(0 for _ in a.shape))
  return pl.pallas_call(
      _tc_body,
      grid=(BATCH // BLK,),
      in_specs=[rows, rows, full(w1u), full(w1i), full(b1), full(w2),
                full(b2), full(wfm), full(wfh), full(bf)],
      out_specs=pl.BlockSpec((BLK, 1), lambda i: (i, 0)),
      out_shape=jax.ShapeDtypeStruct((BATCH, 1), jnp.float32),
  )(u_rows, i_rows, w1u, w1i, b1, w2, b2, wfm, wfh, bf)


def kernel(users, items, user_mf, item_mf, user_mlp, item_mlp,
           W1, b1, W2, b2, Wf, bf):
  u3 = users.reshape(NW, NCH, CHUNK)
  i3 = items.reshape(NW, NCH, CHUNK)
  eyeb = jnp.eye(DF, dtype=jnp.bfloat16)
  eye = jnp.concatenate([eyeb, eyeb], axis=0)
  utab = _tc_fuse(user_mf.T, user_mlp.T, eye)
  itab = _tc_fuse(item_mf.T, item_mlp.T, eye)
  u_rows, i_rows = _sc_gather(u3, i3, utab, itab)
  zeros = jnp.zeros((W1.shape[0], D), jnp.float32)
  w1u = jnp.concatenate([zeros, W1[:, :D]], axis=1)
  w1i = jnp.concatenate([zeros, W1[:, D:]], axis=1)
  wfm = jnp.concatenate([Wf[:, :D], jnp.zeros((1, D), jnp.float32)],
                        axis=1)
  wfh = Wf[:, D:]
  return _tc_mlp(u_rows, i_rows, w1u, w1i, b1.reshape(1, -1),
                 W2, b2.reshape(1, -1), wfm, wfh, bf.reshape(1, 1))
